# baseline (speedup 1.0000x reference)
import functools

import jax
import jax.numpy as jnp
from jax import lax
from jax.experimental import pallas as pl
from jax.experimental.pallas import tpu as pltpu
from jax.experimental.pallas import tpu_sc as plsc

N_NODES = 10000
N_EDGES = 320000
D = 128
HID = 128
OUT = 128

NC = 2
NS = 16
NW = NC * NS
CHUNK = 80
N_CHUNKS = 125
E_PER_W = N_CHUNKS * CHUNK
N_PAIRS = N_CHUNKS // 2
ROWS_PER_TILE = 624
TAIL_ROWS = N_NODES - NS * ROWS_PER_TILE

_mesh = plsc.VectorSubcoreMesh(core_axis_name="c", subcore_axis_name="s",
                               num_cores=NC, num_subcores=NS)


@functools.partial(
    pl.kernel,
    out_type=jax.ShapeDtypeStruct((NC, N_NODES, D), jnp.float32),
    mesh=_mesh,
    scratch_types=[
        pltpu.VMEM((E_PER_W,), jnp.int32),
        pltpu.VMEM((E_PER_W,), jnp.int32),
        pltpu.VMEM((CHUNK, D), jnp.float32),
        pltpu.VMEM((CHUNK, D), jnp.float32),
        pltpu.VMEM_SHARED((N_NODES, D), jnp.float32),
        pltpu.SemaphoreType.DMA,
        pltpu.SemaphoreType.DMA,
    ],
)
def _sc_aggregate(x_hbm, src_hbm, dst_hbm, zeros_hbm, out_hbm,
                  sblk, dblk, rows0, rows1, agg_sh, sem0, sem1):
    c = lax.axis_index("c")
    s = lax.axis_index("s")
    wid = s * NC + c

    pltpu.sync_copy(zeros_hbm.at[pl.ds(0, ROWS_PER_TILE)],
                    agg_sh.at[pl.ds(s * ROWS_PER_TILE, ROWS_PER_TILE)])

    @pl.when(s == 0)
    def _zero_tail():
        pltpu.sync_copy(zeros_hbm.at[pl.ds(0, TAIL_ROWS)],
                        agg_sh.at[pl.ds(NS * ROWS_PER_TILE, TAIL_ROWS)])

    plsc.subcore_barrier()

    base = wid * E_PER_W
    pltpu.sync_copy(src_hbm.at[pl.ds(base, E_PER_W)], sblk)
    pltpu.sync_copy(dst_hbm.at[pl.ds(base, E_PER_W)], dblk)
    pltpu.async_copy(x_hbm.at[sblk.at[pl.ds(0, CHUNK)]], rows0, sem0)

    def pair_body(k, carry2):
        o0 = 2 * k * CHUNK
        pltpu.async_copy(
            x_hbm.at[sblk.at[pl.ds(o0 + CHUNK, CHUNK)]], rows1, sem1)
        pltpu.make_async_copy(
            x_hbm.at[sblk.at[pl.ds(o0, CHUNK)]], rows0, sem0).wait()
        pltpu.sync_copy(rows0, agg_sh.at[dblk.at[pl.ds(o0, CHUNK)]],
                        add=True)
        pltpu.async_copy(
            x_hbm.at[sblk.at[pl.ds(o0 + 2 * CHUNK, CHUNK)]], rows0, sem0)
        pltpu.make_async_copy(
            x_hbm.at[sblk.at[pl.ds(o0 + CHUNK, CHUNK)]], rows1,
            sem1).wait()
        pltpu.sync_copy(rows1,
                        agg_sh.at[dblk.at[pl.ds(o0 + CHUNK, CHUNK)]],
                        add=True)
        return carry2

    lax.fori_loop(0, N_PAIRS, pair_body, 0)

    ot = (N_CHUNKS - 1) * CHUNK
    pltpu.make_async_copy(
        x_hbm.at[sblk.at[pl.ds(ot, CHUNK)]], rows0, sem0).wait()
    pltpu.sync_copy(rows0, agg_sh.at[dblk.at[pl.ds(ot, CHUNK)]], add=True)

    plsc.subcore_barrier()
    pltpu.sync_copy(agg_sh.at[pl.ds(s * ROWS_PER_TILE, ROWS_PER_TILE)],
                    out_hbm.at[c, pl.ds(s * ROWS_PER_TILE, ROWS_PER_TILE)])

    @pl.when(s == 0)
    def _copy_tail():
        pltpu.sync_copy(agg_sh.at[pl.ds(NS * ROWS_PER_TILE, TAIL_ROWS)],
                        out_hbm.at[c, pl.ds(NS * ROWS_PER_TILE, TAIL_ROWS)])


_BLK = 1000


def _mlp_body(eps_ref, x_ref, a0_ref, a1_ref, w1_ref, b1_ref, w2_ref, b2_ref,
              w3_ref, b3_ref, w4_ref, b4_ref, out_ref):
    h = (1.0 + eps_ref[0]) * x_ref[...] + a0_ref[0] + a1_ref[0]
    h = jnp.maximum(
        jnp.dot(h, w1_ref[...], preferred_element_type=jnp.float32)
        + b1_ref[...], 0.0)
    h = jnp.dot(h, w2_ref[...], preferred_element_type=jnp.float32) + b2_ref[...]
    h = jnp.maximum(
        jnp.dot(h, w3_ref[...], preferred_element_type=jnp.float32)
        + b3_ref[...], 0.0)
    h = jnp.dot(h, w4_ref[...], preferred_element_type=jnp.float32) + b4_ref[...]
    out_ref[...] = jax.nn.sigmoid(h)


def _row_spec(i):
    return (i, 0)


def _fixed_spec(i):
    return (0, 0)


_tc_mlp = pl.pallas_call(
    _mlp_body,
    grid=(N_NODES // _BLK,),
    in_specs=[
        pl.BlockSpec(memory_space=pltpu.SMEM),
        pl.BlockSpec((_BLK, D), _row_spec),
        pl.BlockSpec((1, _BLK, D), lambda i: (0, i, 0)),
        pl.BlockSpec((1, _BLK, D), lambda i: (1, i, 0)),
        pl.BlockSpec((D, HID), _fixed_spec),
        pl.BlockSpec((1, HID), _fixed_spec),
        pl.BlockSpec((HID, D), _fixed_spec),
        pl.BlockSpec((1, D), _fixed_spec),
        pl.BlockSpec((D, HID), _fixed_spec),
        pl.BlockSpec((1, HID), _fixed_spec),
        pl.BlockSpec((HID, OUT), _fixed_spec),
        pl.BlockSpec((1, OUT), _fixed_spec),
    ],
    out_specs=pl.BlockSpec((_BLK, OUT), _row_spec),
    out_shape=jax.ShapeDtypeStruct((N_NODES, OUT), jnp.float32),
)


@jax.jit
def kernel(x, edge_index, eps, W1, b1, W2, b2, W3, b3, W4, b4):
    src = edge_index[0].astype(jnp.int32).reshape(N_EDGES)
    dst = edge_index[1].astype(jnp.int32).reshape(N_EDGES)
    zeros = jnp.zeros((ROWS_PER_TILE, D), jnp.float32)
    agg = _sc_aggregate(x, src, dst, zeros)
    return _tc_mlp(jnp.reshape(1.0 * eps, (1,)), x, agg, agg,
                   W1, b1.reshape(1, HID), W2, b2.reshape(1, D),
                   W3, b3.reshape(1, HID), W4, b4.reshape(1, OUT))


# speedup vs baseline: 1.0073x; 1.0073x over previous
import functools

import jax
import jax.numpy as jnp
from jax import lax
from jax.experimental import pallas as pl
from jax.experimental.pallas import tpu as pltpu
from jax.experimental.pallas import tpu_sc as plsc

N_NODES = 10000
N_EDGES = 320000
D = 128
HID = 128
OUT = 128

NC = 2
NS = 16
NW = NC * NS
CHUNK = 80
N_CHUNKS = 125
E_PER_W = N_CHUNKS * CHUNK
N_PAIRS = N_CHUNKS // 2
ROWS_PER_TILE = 624
TAIL_ROWS = N_NODES - NS * ROWS_PER_TILE

_mesh = plsc.VectorSubcoreMesh(core_axis_name="c", subcore_axis_name="s",
                               num_cores=NC, num_subcores=NS)


@functools.partial(
    pl.kernel,
    out_type=jax.ShapeDtypeStruct((NC, N_NODES, D), jnp.float32),
    mesh=_mesh,
    scratch_types=[
        pltpu.VMEM((E_PER_W,), jnp.int32),
        pltpu.VMEM((E_PER_W,), jnp.int32),
        pltpu.VMEM((CHUNK, D), jnp.float32),
        pltpu.VMEM((CHUNK, D), jnp.float32),
        pltpu.VMEM_SHARED((N_NODES, D), jnp.float32),
        pltpu.SemaphoreType.DMA,
        pltpu.SemaphoreType.DMA,
    ],
)
def _sc_aggregate(x_hbm, src_hbm, dst_hbm, zeros_hbm, out_hbm,
                  sblk, dblk, rows0, rows1, agg_sh, sem0, sem1):
    c = lax.axis_index("c")
    s = lax.axis_index("s")
    wid = s * NC + c

    base = wid * E_PER_W
    pltpu.sync_copy(src_hbm.at[pl.ds(base, E_PER_W)], sblk)
    pltpu.sync_copy(dst_hbm.at[pl.ds(base, E_PER_W)], dblk)
    pltpu.async_copy(x_hbm.at[sblk.at[pl.ds(0, CHUNK)]], rows0, sem0)

    pltpu.sync_copy(zeros_hbm.at[pl.ds(0, ROWS_PER_TILE)],
                    agg_sh.at[pl.ds(s * ROWS_PER_TILE, ROWS_PER_TILE)])

    @pl.when(s == 0)
    def _zero_tail():
        pltpu.sync_copy(zeros_hbm.at[pl.ds(0, TAIL_ROWS)],
                        agg_sh.at[pl.ds(NS * ROWS_PER_TILE, TAIL_ROWS)])

    plsc.subcore_barrier()


    def pair_body(k, carry2):
        o0 = 2 * k * CHUNK
        pltpu.async_copy(
            x_hbm.at[sblk.at[pl.ds(o0 + CHUNK, CHUNK)]], rows1, sem1)
        pltpu.make_async_copy(
            x_hbm.at[sblk.at[pl.ds(o0, CHUNK)]], rows0, sem0).wait()
        pltpu.sync_copy(rows0, agg_sh.at[dblk.at[pl.ds(o0, CHUNK)]],
                        add=True)
        pltpu.async_copy(
            x_hbm.at[sblk.at[pl.ds(o0 + 2 * CHUNK, CHUNK)]], rows0, sem0)
        pltpu.make_async_copy(
            x_hbm.at[sblk.at[pl.ds(o0 + CHUNK, CHUNK)]], rows1,
            sem1).wait()
        pltpu.sync_copy(rows1,
                        agg_sh.at[dblk.at[pl.ds(o0 + CHUNK, CHUNK)]],
                        add=True)
        return carry2

    lax.fori_loop(0, N_PAIRS, pair_body, 0)

    ot = (N_CHUNKS - 1) * CHUNK
    pltpu.make_async_copy(
        x_hbm.at[sblk.at[pl.ds(ot, CHUNK)]], rows0, sem0).wait()
    pltpu.sync_copy(rows0, agg_sh.at[dblk.at[pl.ds(ot, CHUNK)]], add=True)

    plsc.subcore_barrier()
    pltpu.sync_copy(agg_sh.at[pl.ds(s * ROWS_PER_TILE, ROWS_PER_TILE)],
                    out_hbm.at[c, pl.ds(s * ROWS_PER_TILE, ROWS_PER_TILE)])

    @pl.when(s == 0)
    def _copy_tail():
        pltpu.sync_copy(agg_sh.at[pl.ds(NS * ROWS_PER_TILE, TAIL_ROWS)],
                        out_hbm.at[c, pl.ds(NS * ROWS_PER_TILE, TAIL_ROWS)])


_BLK = 1000


def _mlp_body(eps_ref, x_ref, a0_ref, a1_ref, w1_ref, b1_ref, w2_ref, b2_ref,
              w3_ref, b3_ref, w4_ref, b4_ref, out_ref):
    h = (1.0 + eps_ref[0]) * x_ref[...] + a0_ref[0] + a1_ref[0]
    h = jnp.maximum(
        jnp.dot(h.astype(jnp.bfloat16), w1_ref[...],
                preferred_element_type=jnp.float32) + b1_ref[...], 0.0)
    h = jnp.dot(h.astype(jnp.bfloat16), w2_ref[...],
                preferred_element_type=jnp.float32) + b2_ref[...]
    h = jnp.maximum(
        jnp.dot(h.astype(jnp.bfloat16), w3_ref[...],
                preferred_element_type=jnp.float32) + b3_ref[...], 0.0)
    h = jnp.dot(h.astype(jnp.bfloat16), w4_ref[...],
                preferred_element_type=jnp.float32) + b4_ref[...]
    out_ref[...] = jax.nn.sigmoid(h)


def _row_spec(i):
    return (i, 0)


def _fixed_spec(i):
    return (0, 0)


_tc_mlp = pl.pallas_call(
    _mlp_body,
    grid=(N_NODES // _BLK,),
    in_specs=[
        pl.BlockSpec(memory_space=pltpu.SMEM),
        pl.BlockSpec((_BLK, D), _row_spec),
        pl.BlockSpec((1, _BLK, D), lambda i: (0, i, 0)),
        pl.BlockSpec((1, _BLK, D), lambda i: (1, i, 0)),
        pl.BlockSpec((D, HID), _fixed_spec),
        pl.BlockSpec((1, HID), _fixed_spec),
        pl.BlockSpec((HID, D), _fixed_spec),
        pl.BlockSpec((1, D), _fixed_spec),
        pl.BlockSpec((D, HID), _fixed_spec),
        pl.BlockSpec((1, HID), _fixed_spec),
        pl.BlockSpec((HID, OUT), _fixed_spec),
        pl.BlockSpec((1, OUT), _fixed_spec),
    ],
    out_specs=pl.BlockSpec((_BLK, OUT), _row_spec),
    out_shape=jax.ShapeDtypeStruct((N_NODES, OUT), jnp.float32),
)


@jax.jit
def kernel(x, edge_index, eps, W1, b1, W2, b2, W3, b3, W4, b4):
    src = edge_index[0].astype(jnp.int32).reshape(N_EDGES)
    dst = edge_index[1].astype(jnp.int32).reshape(N_EDGES)
    zeros = jnp.zeros((ROWS_PER_TILE, D), jnp.float32)
    agg = _sc_aggregate(x, src, dst, zeros)
    bf = jnp.bfloat16
    return _tc_mlp(jnp.reshape(1.0 * eps, (1,)), x, agg, agg,
                   W1.astype(bf), b1.reshape(1, HID),
                   W2.astype(bf), b2.reshape(1, D),
                   W3.astype(bf), b3.reshape(1, HID),
                   W4.astype(bf), b4.reshape(1, OUT))


# speedup vs baseline: 1.0115x; 1.0042x over previous
import functools

import jax
import jax.numpy as jnp
from jax import lax
from jax.experimental import pallas as pl
from jax.experimental.pallas import tpu as pltpu
from jax.experimental.pallas import tpu_sc as plsc

N_NODES = 10000
N_EDGES = 320000
D = 128
HID = 128
OUT = 128

NC = 2
NS = 16
NW = NC * NS
CHUNK = 64
E_PER_W = 10240
E_LAST = N_EDGES - 31 * E_PER_W
N_PAIRS = E_PER_W // (2 * CHUNK)
N_PAIRS_LAST = E_LAST // (2 * CHUNK)
ROWS_PER_TILE = 624
TAIL_ROWS = N_NODES - NS * ROWS_PER_TILE

_mesh = plsc.VectorSubcoreMesh(core_axis_name="c", subcore_axis_name="s",
                               num_cores=NC, num_subcores=NS)


@functools.partial(
    pl.kernel,
    out_type=jax.ShapeDtypeStruct((NC, N_NODES, D), jnp.float32),
    mesh=_mesh,
    scratch_types=[
        pltpu.VMEM((2, E_PER_W), jnp.int32),
        pltpu.VMEM((CHUNK, D), jnp.float32),
        pltpu.VMEM((CHUNK, D), jnp.float32),
        pltpu.VMEM_SHARED((N_NODES, D), jnp.float32),
        pltpu.SemaphoreType.DMA,
        pltpu.SemaphoreType.DMA,
    ],
)
def _sc_aggregate(x_hbm, edge_hbm, zeros_hbm, out_hbm,
                  ebuf, rows0, rows1, agg_sh, sem0, sem1):
    c = lax.axis_index("c")
    s = lax.axis_index("s")
    wid = s * NC + c
    is_last = wid == NW - 1
    n_pairs = jnp.where(is_last, N_PAIRS_LAST, N_PAIRS)

    @pl.when(jnp.logical_not(is_last))
    def _stage_idx():
        pltpu.sync_copy(edge_hbm.at[:, pl.ds(wid * E_PER_W, E_PER_W)], ebuf)

    @pl.when(is_last)
    def _stage_idx_last():
        pltpu.sync_copy(edge_hbm.at[:, pl.ds((NW - 1) * E_PER_W, E_LAST)],
                        ebuf.at[:, pl.ds(0, E_LAST)])

    pltpu.async_copy(x_hbm.at[ebuf.at[0, pl.ds(0, CHUNK)]], rows0, sem0)

    pltpu.sync_copy(zeros_hbm.at[pl.ds(0, ROWS_PER_TILE)],
                    agg_sh.at[pl.ds(s * ROWS_PER_TILE, ROWS_PER_TILE)])

    @pl.when(s == 0)
    def _zero_tail():
        pltpu.sync_copy(zeros_hbm.at[pl.ds(0, TAIL_ROWS)],
                        agg_sh.at[pl.ds(NS * ROWS_PER_TILE, TAIL_ROWS)])

    plsc.subcore_barrier()


    def pair_body(k, carry2):
        o0 = 2 * k * CHUNK
        pltpu.async_copy(
            x_hbm.at[ebuf.at[0, pl.ds(o0 + CHUNK, CHUNK)]], rows1, sem1)
        pltpu.make_async_copy(
            x_hbm.at[ebuf.at[0, pl.ds(o0, CHUNK)]], rows0, sem0).wait()
        pltpu.sync_copy(rows0, agg_sh.at[ebuf.at[1, pl.ds(o0, CHUNK)]],
                        add=True)

        @pl.when(k < n_pairs - 1)
        def _prefetch_next():
            pltpu.async_copy(
                x_hbm.at[ebuf.at[0, pl.ds(o0 + 2 * CHUNK, CHUNK)]],
                rows0, sem0)

        pltpu.make_async_copy(
            x_hbm.at[ebuf.at[0, pl.ds(o0 + CHUNK, CHUNK)]], rows1,
            sem1).wait()
        pltpu.sync_copy(rows1,
                        agg_sh.at[ebuf.at[1, pl.ds(o0 + CHUNK, CHUNK)]],
                        add=True)
        return carry2

    lax.fori_loop(0, n_pairs, pair_body, 0)

    plsc.subcore_barrier()
    pltpu.sync_copy(agg_sh.at[pl.ds(s * ROWS_PER_TILE, ROWS_PER_TILE)],
                    out_hbm.at[c, pl.ds(s * ROWS_PER_TILE, ROWS_PER_TILE)])

    @pl.when(s == 0)
    def _copy_tail():
        pltpu.sync_copy(agg_sh.at[pl.ds(NS * ROWS_PER_TILE, TAIL_ROWS)],
                        out_hbm.at[c, pl.ds(NS * ROWS_PER_TILE, TAIL_ROWS)])


_BLK = 1000


def _mlp_body(eps_ref, x_ref, a0_ref, a1_ref, w1_ref, b1_ref, w2_ref, b2_ref,
              w3_ref, b3_ref, w4_ref, b4_ref, out_ref):
    h = (1.0 + eps_ref[0]) * x_ref[...] + a0_ref[0] + a1_ref[0]
    h = jnp.maximum(
        jnp.dot(h.astype(jnp.bfloat16), w1_ref[...],
                preferred_element_type=jnp.float32) + b1_ref[...], 0.0)
    h = jnp.dot(h.astype(jnp.bfloat16), w2_ref[...],
                preferred_element_type=jnp.float32) + b2_ref[...]
    h = jnp.maximum(
        jnp.dot(h.astype(jnp.bfloat16), w3_ref[...],
                preferred_element_type=jnp.float32) + b3_ref[...], 0.0)
    h = jnp.dot(h.astype(jnp.bfloat16), w4_ref[...],
                preferred_element_type=jnp.float32) + b4_ref[...]
    out_ref[...] = jax.nn.sigmoid(h)


def _row_spec(i):
    return (i, 0)


def _fixed_spec(i):
    return (0, 0)


_tc_mlp = pl.pallas_call(
    _mlp_body,
    grid=(N_NODES // _BLK,),
    in_specs=[
        pl.BlockSpec(memory_space=pltpu.SMEM),
        pl.BlockSpec((_BLK, D), _row_spec),
        pl.BlockSpec((1, _BLK, D), lambda i: (0, i, 0)),
        pl.BlockSpec((1, _BLK, D), lambda i: (1, i, 0)),
        pl.BlockSpec((D, HID), _fixed_spec),
        pl.BlockSpec((1, HID), _fixed_spec),
        pl.BlockSpec((HID, D), _fixed_spec),
        pl.BlockSpec((1, D), _fixed_spec),
        pl.BlockSpec((D, HID), _fixed_spec),
        pl.BlockSpec((1, HID), _fixed_spec),
        pl.BlockSpec((HID, OUT), _fixed_spec),
        pl.BlockSpec((1, OUT), _fixed_spec),
    ],
    out_specs=pl.BlockSpec((_BLK, OUT), _row_spec),
    out_shape=jax.ShapeDtypeStruct((N_NODES, OUT), jnp.float32),
)


@jax.jit
def kernel(x, edge_index, eps, W1, b1, W2, b2, W3, b3, W4, b4):
    zeros = jnp.zeros((ROWS_PER_TILE, D), jnp.float32)
    agg = _sc_aggregate(x, edge_index.astype(jnp.int32), zeros)
    bf = jnp.bfloat16
    return _tc_mlp(jnp.reshape(1.0 * eps, (1,)), x, agg, agg,
                   W1.astype(bf), b1.reshape(1, HID),
                   W2.astype(bf), b2.reshape(1, D),
                   W3.astype(bf), b3.reshape(1, HID),
                   W4.astype(bf), b4.reshape(1, OUT))


# speedup vs baseline: 1.1598x; 1.1466x over previous
import functools

import jax
import jax.numpy as jnp
from jax import lax
from jax.experimental import pallas as pl
from jax.experimental.pallas import tpu as pltpu
from jax.experimental.pallas import tpu_sc as plsc

N_NODES = 10000
N_EDGES = 320000
D = 128
HID = 128
OUT = 128

NC = 2
NS = 16
NW = NC * NS
CHUNK = 128
E_PER_W = 10240
E_LAST = N_EDGES - 31 * E_PER_W
E_HALF = E_PER_W // 2
N_PAIRS = E_HALF // (2 * CHUNK)
N_PAIRS_LAST = E_LAST // (2 * CHUNK)
ROWS_PER_TILE = 624
TAIL_ROWS = N_NODES - NS * ROWS_PER_TILE

_mesh = plsc.VectorSubcoreMesh(core_axis_name="c", subcore_axis_name="s",
                               num_cores=NC, num_subcores=NS)


@functools.partial(
    pl.kernel,
    out_type=jax.ShapeDtypeStruct((NC, N_NODES, D), jnp.float32),
    mesh=_mesh,
    scratch_types=[
        pltpu.VMEM((2, E_HALF), jnp.int32),
        pltpu.VMEM((CHUNK, D), jnp.float32),
        pltpu.VMEM((CHUNK, D), jnp.float32),
        pltpu.VMEM_SHARED((N_NODES, D), jnp.float32),
        pltpu.SemaphoreType.DMA,
        pltpu.SemaphoreType.DMA,
    ],
)
def _sc_aggregate(x_hbm, edge_hbm, zeros_hbm, out_hbm,
                  ebuf, rows0, rows1, agg_sh, sem0, sem1):
    c = lax.axis_index("c")
    s = lax.axis_index("s")
    wid = s * NC + c
    is_last = wid == NW - 1
    n_pairs = jnp.where(is_last, N_PAIRS_LAST, N_PAIRS)

    def run_pairs(n):
        def pair_body(k, carry2):
            o0 = 2 * k * CHUNK
            pltpu.async_copy(
                x_hbm.at[ebuf.at[0, pl.ds(o0 + CHUNK, CHUNK)]], rows1, sem1)
            pltpu.make_async_copy(
                x_hbm.at[ebuf.at[0, pl.ds(o0, CHUNK)]], rows0, sem0).wait()
            pltpu.sync_copy(rows0, agg_sh.at[ebuf.at[1, pl.ds(o0, CHUNK)]],
                            add=True)

            @pl.when(k < n - 1)
            def _prefetch_next():
                pltpu.async_copy(
                    x_hbm.at[ebuf.at[0, pl.ds(o0 + 2 * CHUNK, CHUNK)]],
                    rows0, sem0)

            pltpu.make_async_copy(
                x_hbm.at[ebuf.at[0, pl.ds(o0 + CHUNK, CHUNK)]], rows1,
                sem1).wait()
            pltpu.sync_copy(rows1,
                            agg_sh.at[ebuf.at[1, pl.ds(o0 + CHUNK, CHUNK)]],
                            add=True)
            return carry2

        lax.fori_loop(0, n, pair_body, 0)

    @pl.when(jnp.logical_not(is_last))
    def _stage_idx():
        pltpu.sync_copy(edge_hbm.at[:, pl.ds(wid * E_PER_W, E_HALF)], ebuf)

    @pl.when(is_last)
    def _stage_idx_last():
        pltpu.sync_copy(edge_hbm.at[:, pl.ds((NW - 1) * E_PER_W, E_LAST)],
                        ebuf.at[:, pl.ds(0, E_LAST)])

    pltpu.async_copy(x_hbm.at[ebuf.at[0, pl.ds(0, CHUNK)]], rows0, sem0)

    pltpu.sync_copy(zeros_hbm.at[pl.ds(0, ROWS_PER_TILE)],
                    agg_sh.at[pl.ds(s * ROWS_PER_TILE, ROWS_PER_TILE)])

    @pl.when(s == 0)
    def _zero_tail():
        pltpu.sync_copy(zeros_hbm.at[pl.ds(0, TAIL_ROWS)],
                        agg_sh.at[pl.ds(NS * ROWS_PER_TILE, TAIL_ROWS)])

    plsc.subcore_barrier()

    run_pairs(n_pairs)

    @pl.when(jnp.logical_not(is_last))
    def _second_half():
        pltpu.sync_copy(
            edge_hbm.at[:, pl.ds(wid * E_PER_W + E_HALF, E_HALF)], ebuf)
        pltpu.async_copy(x_hbm.at[ebuf.at[0, pl.ds(0, CHUNK)]], rows0, sem0)
        run_pairs(N_PAIRS)

    plsc.subcore_barrier()
    pltpu.sync_copy(agg_sh.at[pl.ds(s * ROWS_PER_TILE, ROWS_PER_TILE)],
                    out_hbm.at[c, pl.ds(s * ROWS_PER_TILE, ROWS_PER_TILE)])

    @pl.when(s == 0)
    def _copy_tail():
        pltpu.sync_copy(agg_sh.at[pl.ds(NS * ROWS_PER_TILE, TAIL_ROWS)],
                        out_hbm.at[c, pl.ds(NS * ROWS_PER_TILE, TAIL_ROWS)])


_BLK = 1000


def _mlp_body(eps_ref, x_ref, a0_ref, a1_ref, w1_ref, b1_ref, w2_ref, b2_ref,
              w3_ref, b3_ref, w4_ref, b4_ref, out_ref):
    h = (1.0 + eps_ref[0]) * x_ref[...] + a0_ref[0] + a1_ref[0]
    h = jnp.maximum(
        jnp.dot(h.astype(jnp.bfloat16), w1_ref[...],
                preferred_element_type=jnp.float32) + b1_ref[...], 0.0)
    h = jnp.dot(h.astype(jnp.bfloat16), w2_ref[...],
                preferred_element_type=jnp.float32) + b2_ref[...]
    h = jnp.maximum(
        jnp.dot(h.astype(jnp.bfloat16), w3_ref[...],
                preferred_element_type=jnp.float32) + b3_ref[...], 0.0)
    h = jnp.dot(h.astype(jnp.bfloat16), w4_ref[...],
                preferred_element_type=jnp.float32) + b4_ref[...]
    out_ref[...] = jax.nn.sigmoid(h)


def _row_spec(i):
    return (i, 0)


def _fixed_spec(i):
    return (0, 0)


_tc_mlp = pl.pallas_call(
    _mlp_body,
    grid=(N_NODES // _BLK,),
    in_specs=[
        pl.BlockSpec(memory_space=pltpu.SMEM),
        pl.BlockSpec((_BLK, D), _row_spec),
        pl.BlockSpec((1, _BLK, D), lambda i: (0, i, 0)),
        pl.BlockSpec((1, _BLK, D), lambda i: (1, i, 0)),
        pl.BlockSpec((D, HID), _fixed_spec),
        pl.BlockSpec((1, HID), _fixed_spec),
        pl.BlockSpec((HID, D), _fixed_spec),
        pl.BlockSpec((1, D), _fixed_spec),
        pl.BlockSpec((D, HID), _fixed_spec),
        pl.BlockSpec((1, HID), _fixed_spec),
        pl.BlockSpec((HID, OUT), _fixed_spec),
        pl.BlockSpec((1, OUT), _fixed_spec),
    ],
    out_specs=pl.BlockSpec((_BLK, OUT), _row_spec),
    out_shape=jax.ShapeDtypeStruct((N_NODES, OUT), jnp.float32),
)


@jax.jit
def kernel(x, edge_index, eps, W1, b1, W2, b2, W3, b3, W4, b4):
    zeros = jnp.zeros((ROWS_PER_TILE, D), jnp.float32)
    agg = _sc_aggregate(x, edge_index.astype(jnp.int32), zeros)
    bf = jnp.bfloat16
    return _tc_mlp(jnp.reshape(1.0 * eps, (1,)), x, agg, agg,
                   W1.astype(bf), b1.reshape(1, HID),
                   W2.astype(bf), b2.reshape(1, D),
                   W3.astype(bf), b3.reshape(1, HID),
                   W4.astype(bf), b4.reshape(1, OUT))


# speedup vs baseline: 1.1802x; 1.0175x over previous
import functools

import jax
import jax.numpy as jnp
from jax import lax
from jax.experimental import pallas as pl
from jax.experimental.pallas import tpu as pltpu
from jax.experimental.pallas import tpu_sc as plsc

N_NODES = 10000
N_EDGES = 320000
D = 128
HID = 128
OUT = 128

NC = 2
NS = 16
NW = NC * NS
CHUNK = 128
E_PER_W = 10240
E_LAST = N_EDGES - 31 * E_PER_W
E_HALF = E_PER_W // 2
N_PAIRS = E_HALF // (2 * CHUNK)
N_PAIRS_LAST = E_LAST // (2 * CHUNK)
ROWS_PER_TILE = 624
TAIL_ROWS = N_NODES - NS * ROWS_PER_TILE

_mesh = plsc.VectorSubcoreMesh(core_axis_name="c", subcore_axis_name="s",
                               num_cores=NC, num_subcores=NS)


@functools.partial(
    pl.kernel,
    out_type=jax.ShapeDtypeStruct((NC, N_NODES, D), jnp.float32),
    mesh=_mesh,
    scratch_types=[
        pltpu.VMEM((2, E_HALF), jnp.int32),
        pltpu.VMEM((CHUNK, D), jnp.float32),
        pltpu.VMEM((CHUNK, D), jnp.float32),
        pltpu.VMEM_SHARED((N_NODES, D), jnp.float32),
        pltpu.SemaphoreType.DMA,
        pltpu.SemaphoreType.DMA,
    ],
)
def _sc_aggregate(x_hbm, edge_hbm, zeros_hbm, out_hbm,
                  ebuf, rows0, rows1, agg_sh, sem0, sem1):
    c = lax.axis_index("c")
    s = lax.axis_index("s")
    wid = s * NC + c
    is_last = wid == NW - 1
    n_pairs = jnp.where(is_last, N_PAIRS_LAST, N_PAIRS)

    def run_pairs(n):
        def pair_body(k, carry2):
            o0 = 2 * k * CHUNK
            pltpu.async_copy(
                x_hbm.at[ebuf.at[0, pl.ds(o0 + CHUNK, CHUNK)]], rows1, sem1)
            pltpu.make_async_copy(
                x_hbm.at[ebuf.at[0, pl.ds(o0, CHUNK)]], rows0, sem0).wait()
            pltpu.sync_copy(rows0, agg_sh.at[ebuf.at[1, pl.ds(o0, CHUNK)]],
                            add=True)

            @pl.when(k < n - 1)
            def _prefetch_next():
                pltpu.async_copy(
                    x_hbm.at[ebuf.at[0, pl.ds(o0 + 2 * CHUNK, CHUNK)]],
                    rows0, sem0)

            pltpu.make_async_copy(
                x_hbm.at[ebuf.at[0, pl.ds(o0 + CHUNK, CHUNK)]], rows1,
                sem1).wait()
            pltpu.sync_copy(rows1,
                            agg_sh.at[ebuf.at[1, pl.ds(o0 + CHUNK, CHUNK)]],
                            add=True)
            return carry2

        lax.fori_loop(0, n, pair_body, 0)

    @pl.when(jnp.logical_not(is_last))
    def _stage_idx():
        pltpu.sync_copy(edge_hbm.at[:, pl.ds(wid * E_PER_W, E_HALF)], ebuf)

    @pl.when(is_last)
    def _stage_idx_last():
        pltpu.sync_copy(edge_hbm.at[:, pl.ds((NW - 1) * E_PER_W, E_LAST)],
                        ebuf.at[:, pl.ds(0, E_LAST)])

    pltpu.async_copy(x_hbm.at[ebuf.at[0, pl.ds(0, CHUNK)]], rows0, sem0)

    pltpu.sync_copy(zeros_hbm.at[pl.ds(0, ROWS_PER_TILE)],
                    agg_sh.at[pl.ds(s * ROWS_PER_TILE, ROWS_PER_TILE)])

    @pl.when(s == 0)
    def _zero_tail():
        pltpu.sync_copy(zeros_hbm.at[pl.ds(0, TAIL_ROWS)],
                        agg_sh.at[pl.ds(NS * ROWS_PER_TILE, TAIL_ROWS)])

    plsc.subcore_barrier()

    run_pairs(n_pairs)

    @pl.when(jnp.logical_not(is_last))
    def _second_half():
        pltpu.sync_copy(
            edge_hbm.at[:, pl.ds(wid * E_PER_W + E_HALF, E_HALF)], ebuf)
        pltpu.async_copy(x_hbm.at[ebuf.at[0, pl.ds(0, CHUNK)]], rows0, sem0)
        run_pairs(N_PAIRS)

    plsc.subcore_barrier()
    pltpu.sync_copy(agg_sh.at[pl.ds(s * ROWS_PER_TILE, ROWS_PER_TILE)],
                    out_hbm.at[c, pl.ds(s * ROWS_PER_TILE, ROWS_PER_TILE)])

    @pl.when(s == 0)
    def _copy_tail():
        pltpu.sync_copy(agg_sh.at[pl.ds(NS * ROWS_PER_TILE, TAIL_ROWS)],
                        out_hbm.at[c, pl.ds(NS * ROWS_PER_TILE, TAIL_ROWS)])


_BLK = 2000


def _mlp_body(eps_ref, x_ref, a0_ref, a1_ref, w1_ref, b1_ref, w2_ref, b2_ref,
              w3_ref, b3_ref, w4_ref, b4_ref, out_ref):
    h = (1.0 + eps_ref[0]) * x_ref[...] + a0_ref[0] + a1_ref[0]
    h = jnp.maximum(
        jnp.dot(h.astype(jnp.bfloat16), w1_ref[...],
                preferred_element_type=jnp.float32) + b1_ref[...], 0.0)
    h = jnp.dot(h.astype(jnp.bfloat16), w2_ref[...],
                preferred_element_type=jnp.float32) + b2_ref[...]
    h = jnp.maximum(
        jnp.dot(h.astype(jnp.bfloat16), w3_ref[...],
                preferred_element_type=jnp.float32) + b3_ref[...], 0.0)
    h = jnp.dot(h.astype(jnp.bfloat16), w4_ref[...],
                preferred_element_type=jnp.float32) + b4_ref[...]
    out_ref[...] = jax.nn.sigmoid(h)


def _row_spec(i):
    return (i, 0)


def _fixed_spec(i):
    return (0, 0)


_tc_mlp = pl.pallas_call(
    _mlp_body,
    grid=(N_NODES // _BLK,),
    in_specs=[
        pl.BlockSpec(memory_space=pltpu.SMEM),
        pl.BlockSpec((_BLK, D), _row_spec),
        pl.BlockSpec((1, _BLK, D), lambda i: (0, i, 0)),
        pl.BlockSpec((1, _BLK, D), lambda i: (1, i, 0)),
        pl.BlockSpec((D, HID), _fixed_spec),
        pl.BlockSpec((1, HID), _fixed_spec),
        pl.BlockSpec((HID, D), _fixed_spec),
        pl.BlockSpec((1, D), _fixed_spec),
        pl.BlockSpec((D, HID), _fixed_spec),
        pl.BlockSpec((1, HID), _fixed_spec),
        pl.BlockSpec((HID, OUT), _fixed_spec),
        pl.BlockSpec((1, OUT), _fixed_spec),
    ],
    out_specs=pl.BlockSpec((_BLK, OUT), _row_spec),
    out_shape=jax.ShapeDtypeStruct((N_NODES, OUT), jnp.float32),
)


@jax.jit
def kernel(x, edge_index, eps, W1, b1, W2, b2, W3, b3, W4, b4):
    zeros = jnp.zeros((ROWS_PER_TILE, D), jnp.float32)
    agg = _sc_aggregate(x, edge_index.astype(jnp.int32), zeros)
    bf = jnp.bfloat16
    return _tc_mlp(jnp.reshape(1.0 * eps, (1,)), x, agg, agg,
                   W1.astype(bf), b1.reshape(1, HID),
                   W2.astype(bf), b2.reshape(1, D),
                   W3.astype(bf), b3.reshape(1, HID),
                   W4.astype(bf), b4.reshape(1, OUT))
